# baseline (speedup 1.0000x reference)
import functools

import jax
import jax.numpy as jnp
from jax import lax
from jax.experimental import pallas as pl
from jax.experimental.pallas import tpu as pltpu
from jax.experimental.pallas import tpu_sc as plsc

NC = 2
NS = 16
NW = NC * NS
C = 128
R = 2


def _ceil_to(x, m):
    return (x + m - 1) // m * m


@functools.lru_cache(maxsize=None)
def _make_stage1(N, D, NP, EPW):
    NCH = EPW // C
    RPT = NP // NS
    NZ = RPT // C
    mesh = plsc.VectorSubcoreMesh(core_axis_name="c", subcore_axis_name="s")

    @functools.partial(
        pl.kernel,
        out_type=[
            jax.ShapeDtypeStruct((NC * NP, D), jnp.float32),
            jax.ShapeDtypeStruct((NC * NP,), jnp.float32),
            jax.ShapeDtypeStruct((NC * NP,), jnp.float32),
        ],
        mesh=mesh,
        scratch_types=[
            pltpu.VMEM((C,), jnp.int32),
            pltpu.VMEM((C,), jnp.int32),
            pltpu.VMEM((C,), jnp.int32),
            pltpu.VMEM((C,), jnp.int32),
            pltpu.VMEM((C,), jnp.int32),
            pltpu.VMEM((C,), jnp.int32),
            pltpu.VMEM((C, D), jnp.float32),
            pltpu.VMEM((C, D), jnp.float32),
            pltpu.VMEM((C,), jnp.float32),
            pltpu.VMEM((RPT,), jnp.float32),
            pltpu.VMEM_SHARED((NP, D), jnp.float32),
            pltpu.VMEM_SHARED((NP,), jnp.float32),
            pltpu.VMEM_SHARED((NP,), jnp.float32),
            [pltpu.SemaphoreType.DMA] * R,
            pltpu.SemaphoreType.DMA,
        ],
    )
    def stage1(x_hbm, srcg_hbm, dsts_hbm, srcs_hbm,
               agg_out, deg_out, cnt_out,
               idxg0, idxg1, idxd0, idxd1, idxs0, idxs1, buf0, buf1,
               ones1, hbounce, agg_sh, deg_sh, cnt_sh, gsems, hsem):
        cid = lax.axis_index("c")
        sid = lax.axis_index("s")
        wid = sid * NC + cid
        zero16 = jnp.zeros((16,), jnp.float32)
        one16 = jnp.ones((16,), jnp.float32)
        bufl = [buf0, buf1]
        idxg = [idxg0, idxg1]
        idxd = [idxd0, idxd1]
        idxs = [idxs0, idxs1]
        rows = buf0

        def _zrow(r, _):
            for k in range(D // 16):
                rows[r, pl.ds(k * 16, 16)] = zero16
            return 0

        lax.fori_loop(0, C, _zrow, 0)
        for k in range(C // 16):
            ones1[pl.ds(k * 16, 16)] = one16

        def _zh(r, _):
            hbounce[pl.ds(r * 16, 16)] = zero16
            return 0

        lax.fori_loop(0, RPT // 16, _zh, 0)
        r0 = sid * RPT

        def _zcp(t, _):
            pltpu.sync_copy(rows, agg_sh.at[pl.ds(r0 + t * C, C)])
            return 0

        lax.fori_loop(0, NZ, _zcp, 0)
        pltpu.sync_copy(hbounce, deg_sh.at[pl.ds(r0, RPT)])
        pltpu.sync_copy(hbounce, cnt_sh.at[pl.ds(r0, RPT)])
        plsc.subcore_barrier()

        ebase = wid * EPW
        for b in range(R):
            pltpu.sync_copy(srcg_hbm.at[pl.ds(ebase + b * C, C)], idxg[b])
            pltpu.async_copy(x_hbm.at[idxg[b]], bufl[b], gsems[b])

        def _group(g, _):
            for b in range(R):
                j = g * R + b
                pltpu.make_async_copy(x_hbm.at[idxg[b]], bufl[b],
                                      gsems[b]).wait()
                pltpu.sync_copy(dsts_hbm.at[pl.ds(ebase + j * C, C)], idxd[b])
                pltpu.sync_copy(srcs_hbm.at[pl.ds(ebase + j * C, C)], idxs[b])
                hd = pltpu.async_copy(ones1, deg_sh.at[idxd[b]], hsem,
                                      add=True)
                hc = pltpu.async_copy(ones1, cnt_sh.at[idxs[b]], hsem,
                                      add=True)
                pltpu.sync_copy(bufl[b], agg_sh.at[idxd[b]], add=True)
                hd.wait()
                hc.wait()
                jw = lax.rem(j + R, NCH)
                pltpu.sync_copy(srcg_hbm.at[pl.ds(ebase + jw * C, C)], idxg[b])
                pltpu.async_copy(x_hbm.at[idxg[b]], bufl[b], gsems[b])
            return 0

        lax.fori_loop(0, NCH // R, _group, 0)
        for b in range(R):
            pltpu.make_async_copy(x_hbm.at[idxg[b]], bufl[b], gsems[b]).wait()
        plsc.subcore_barrier()

        def _ocp(t, _):
            pltpu.sync_copy(agg_sh.at[pl.ds(r0 + t * C, C)], rows)
            pltpu.sync_copy(rows, agg_out.at[pl.ds(cid * NP + r0 + t * C, C)])
            return 0

        lax.fori_loop(0, NZ, _ocp, 0)
        pltpu.sync_copy(deg_sh.at[pl.ds(r0, RPT)], hbounce)
        pltpu.sync_copy(hbounce, deg_out.at[pl.ds(cid * NP + r0, RPT)])
        pltpu.sync_copy(cnt_sh.at[pl.ds(r0, RPT)], hbounce)
        pltpu.sync_copy(hbounce, cnt_out.at[pl.ds(cid * NP + r0, RPT)])

    return stage1


@functools.lru_cache(maxsize=None)
def _make_stage2(N, D, NP, EPW):
    C2 = C // 2
    NCH = EPW // C2
    RPT = NP // NS
    NZ = RPT // C2
    mesh = plsc.VectorSubcoreMesh(core_axis_name="c", subcore_axis_name="s")

    @functools.partial(
        pl.kernel,
        out_type=jax.ShapeDtypeStruct((NC * NP, D), jnp.float32),
        mesh=mesh,
        scratch_types=[
            pltpu.VMEM((C2,), jnp.int32),
            pltpu.VMEM((C2,), jnp.int32),
            pltpu.VMEM((C2,), jnp.int32),
            pltpu.VMEM((C2,), jnp.int32),
            pltpu.VMEM((C2,), jnp.int32),
            pltpu.VMEM((C2,), jnp.int32),
            pltpu.VMEM((C2, D), jnp.float32),
            pltpu.VMEM((C2, D), jnp.float32),
            pltpu.VMEM((C2, D), jnp.float32),
            pltpu.VMEM((C2, D), jnp.float32),
            pltpu.VMEM_SHARED((NP, D), jnp.float32),
            [pltpu.SemaphoreType.DMA] * R,
            [pltpu.SemaphoreType.DMA] * R,
        ],
    )
    def stage2(h_hbm, srcg_hbm, dstg_hbm, srcs_hbm,
               s_out,
               idxa0, idxa1, idxb0, idxb1, idxs0, idxs1,
               bufa0, bufa1, bufb0, bufb1,
               s_sh, sems_a, sems_b):
        cid = lax.axis_index("c")
        sid = lax.axis_index("s")
        wid = sid * NC + cid
        zero16 = jnp.zeros((16,), jnp.float32)
        bufa = [bufa0, bufa1]
        bufb = [bufb0, bufb1]
        idxa = [idxa0, idxa1]
        idxb = [idxb0, idxb1]
        idxs = [idxs0, idxs1]

        def _zrow(r, _):
            for k in range(D // 16):
                bufa0[r, pl.ds(k * 16, 16)] = zero16
            return 0

        lax.fori_loop(0, C2, _zrow, 0)
        r0 = sid * RPT

        def _zcp(t, _):
            pltpu.sync_copy(bufa0, s_sh.at[pl.ds(r0 + t * C2, C2)])
            return 0

        lax.fori_loop(0, NZ, _zcp, 0)
        plsc.subcore_barrier()

        ebase = wid * EPW
        for b in range(R):
            pltpu.sync_copy(srcg_hbm.at[pl.ds(ebase + b * C2, C2)], idxa[b])
            pltpu.sync_copy(dstg_hbm.at[pl.ds(ebase + b * C2, C2)], idxb[b])
            pltpu.async_copy(h_hbm.at[idxa[b]], bufa[b], sems_a[b])
            pltpu.async_copy(h_hbm.at[idxb[b]], bufb[b], sems_b[b])

        def _group(g, _):
            for b in range(R):
                j = g * R + b
                pltpu.make_async_copy(h_hbm.at[idxa[b]], bufa[b],
                                      sems_a[b]).wait()
                pltpu.make_async_copy(h_hbm.at[idxb[b]], bufb[b],
                                      sems_b[b]).wait()
                pltpu.sync_copy(srcs_hbm.at[pl.ds(ebase + j * C2, C2)], idxs[b])

                def _erow(r, _):
                    for k in range(D // 16):
                        a = bufa[b][r, pl.ds(k * 16, 16)]
                        bb = bufb[b][r, pl.ds(k * 16, 16)]
                        d = a - bb
                        bufa[b][r, pl.ds(k * 16, 16)] = d * d
                    return 0

                lax.fori_loop(0, C, _erow, 0)
                pltpu.sync_copy(bufa[b], s_sh.at[idxs[b]], add=True)
                jw = lax.rem(j + R, NCH)
                pltpu.sync_copy(srcg_hbm.at[pl.ds(ebase + jw * C2, C2)], idxa[b])
                pltpu.sync_copy(dstg_hbm.at[pl.ds(ebase + jw * C2, C2)], idxb[b])
                pltpu.async_copy(h_hbm.at[idxa[b]], bufa[b], sems_a[b])
                pltpu.async_copy(h_hbm.at[idxb[b]], bufb[b], sems_b[b])
            return 0

        lax.fori_loop(0, NCH // R, _group, 0)
        for b in range(R):
            pltpu.make_async_copy(h_hbm.at[idxa[b]], bufa[b],
                                  sems_a[b]).wait()
            pltpu.make_async_copy(h_hbm.at[idxb[b]], bufb[b],
                                  sems_b[b]).wait()
        plsc.subcore_barrier()

        def _ocp(t, _):
            pltpu.sync_copy(s_sh.at[pl.ds(r0 + t * C2, C2)], bufa0)
            pltpu.sync_copy(bufa0, s_out.at[pl.ds(cid * NP + r0 + t * C2, C2)])
            return 0

        lax.fori_loop(0, NZ, _ocp, 0)

    return stage2


@functools.lru_cache(maxsize=None)
def _make_conv(N, D, BN):
    def body(x_ref, agg_ref, d0_ref, d1_ref, ws_ref, wn_ref, b_ref, h_ref):
        deg = jnp.maximum(d0_ref[...] + d1_ref[...], 1.0)
        mean = (agg_ref[0] + agg_ref[1]) / deg
        h = jnp.dot(x_ref[...], ws_ref[...], preferred_element_type=jnp.float32)
        h = h + jnp.dot(mean, wn_ref[...], preferred_element_type=jnp.float32)
        h = h + b_ref[...]
        h_ref[...] = jnp.maximum(h, 0.0)

    return pl.pallas_call(
        body,
        grid=(N // BN,),
        in_specs=[
            pl.BlockSpec((BN, D), lambda i: (i, 0)),
            pl.BlockSpec((NC, BN, D), lambda i: (0, i, 0)),
            pl.BlockSpec((BN, 1), lambda i: (i, 0)),
            pl.BlockSpec((BN, 1), lambda i: (i, 0)),
            pl.BlockSpec((D, D), lambda i: (0, 0)),
            pl.BlockSpec((D, D), lambda i: (0, 0)),
            pl.BlockSpec((1, D), lambda i: (0, 0)),
        ],
        out_specs=pl.BlockSpec((BN, D), lambda i: (i, 0)),
        out_shape=jax.ShapeDtypeStruct((N, D), jnp.float32),
    )


@functools.lru_cache(maxsize=None)
def _make_final(N, D, BN):
    def body(s_ref, c0_ref, c1_ref, gg_ref):
        cnt = jnp.maximum(c0_ref[...] + c1_ref[...], 1.0)
        gg_ref[...] = jnp.tanh((s_ref[0] + s_ref[1]) / cnt)

    return pl.pallas_call(
        body,
        grid=(N // BN,),
        in_specs=[
            pl.BlockSpec((NC, BN, D), lambda i: (0, i, 0)),
            pl.BlockSpec((BN, 1), lambda i: (i, 0)),
            pl.BlockSpec((BN, 1), lambda i: (i, 0)),
        ],
        out_specs=pl.BlockSpec((BN, D), lambda i: (i, 0)),
        out_shape=jax.ShapeDtypeStruct((N, D), jnp.float32),
    )


def kernel(X, edge_index, W_self, W_neigh, b):
    N, D = X.shape
    E = edge_index.shape[1]
    NP = _ceil_to(N + 1, NS * C)
    E_pad = _ceil_to(E, NW * C * R)
    EPW = E_pad // NW

    src = edge_index[0]
    dst = edge_index[1]
    pad = E_pad - E
    zpad = jnp.zeros((pad,), jnp.int32)
    gpad = jnp.full((pad,), N, jnp.int32)
    src_g = jnp.concatenate([src, zpad])
    dst_g = jnp.concatenate([dst, zpad])
    src_s = jnp.concatenate([src, gpad])
    dst_s = jnp.concatenate([dst, gpad])

    agg2, deg2, cnt2 = _make_stage1(N, D, NP, EPW)(X, src_g, dst_s, src_s)
    agg2 = agg2.reshape(NC, NP, D)
    d0 = deg2[:N, None]
    d1 = deg2[NP:NP + N, None]
    c0 = cnt2[:N, None]
    c1 = cnt2[NP:NP + N, None]
    h = _make_conv(N, D, 400)(X, agg2, d0, d1, W_self, W_neigh,
                              b.reshape(1, D))
    s2 = _make_stage2(N, D, NP, EPW)(h, src_g, dst_g, src_s)
    s2 = s2.reshape(NC, NP, D)
    gg = _make_final(N, D, 400)(s2, c0, c1)
    return gg


# speedup vs baseline: 1.3268x; 1.3268x over previous
import functools

import jax
import jax.numpy as jnp
from jax import lax
from jax.experimental import pallas as pl
from jax.experimental.pallas import tpu as pltpu
from jax.experimental.pallas import tpu_sc as plsc

NC = 2
NS = 16
NW = NC * NS
C = 128


def _ceil_to(x, m):
    return (x + m - 1) // m * m


@functools.lru_cache(maxsize=None)
def _make_stage1(N, D, NP, EPW, EPT):
    nchunks = EPW // C
    hchunks = EPT // C
    RPT = NP // NS
    NZ = RPT // C
    mesh = plsc.VectorSubcoreMesh(core_axis_name="c", subcore_axis_name="s")

    @functools.partial(
        pl.kernel,
        out_type=[
            jax.ShapeDtypeStruct((NC * NP, D), jnp.float32),
            jax.ShapeDtypeStruct((NC * NP,), jnp.float32),
        ],
        mesh=mesh,
        scratch_types=[
            pltpu.VMEM((C,), jnp.int32),
            pltpu.VMEM((C,), jnp.int32),
            pltpu.VMEM((C, D), jnp.float32),
            pltpu.VMEM((C,), jnp.float32),
            pltpu.VMEM((RPT,), jnp.float32),
            pltpu.VMEM_SHARED((NP, D), jnp.float32),
            pltpu.VMEM_SHARED((NP,), jnp.float32),
            pltpu.SemaphoreType.DMA,
            pltpu.SemaphoreType.DMA,
        ],
    )
    def stage1(x_hbm, srcg_hbm, dsts_hbm, histidx_hbm,
               agg_out, hist_out,
               idx_g, idx_d, rows, ones1, hbounce,
               agg_sh, hist_sh, sem, isem):
        cid = lax.axis_index("c")
        sid = lax.axis_index("s")
        wid = sid * NC + cid
        zero16 = jnp.zeros((16,), jnp.float32)
        one16 = jnp.ones((16,), jnp.float32)

        def _zrow(r, _):
            for k in range(D // 16):
                rows[r, pl.ds(k * 16, 16)] = zero16
            return 0

        lax.fori_loop(0, C, _zrow, 0)
        for k in range(C // 16):
            ones1[pl.ds(k * 16, 16)] = one16

        def _zh(r, _):
            hbounce[pl.ds(r * 16, 16)] = zero16
            return 0

        lax.fori_loop(0, RPT // 16, _zh, 0)

        r0 = sid * RPT
        for t in range(NZ):
            pltpu.sync_copy(rows, agg_sh.at[pl.ds(r0 + t * C, C)])
        pltpu.sync_copy(hbounce, hist_sh.at[pl.ds(r0, RPT)])
        plsc.subcore_barrier()

        def _chunk(j, _):
            base = wid * EPW + j * C
            ca = pltpu.async_copy(srcg_hbm.at[pl.ds(base, C)], idx_g, isem)
            cb = pltpu.async_copy(dsts_hbm.at[pl.ds(base, C)], idx_d, isem)
            ca.wait()
            cb.wait()
            pltpu.async_copy(x_hbm.at[idx_g], rows, sem).wait()
            pltpu.sync_copy(rows, agg_sh.at[idx_d], add=True)
            return 0

        lax.fori_loop(0, nchunks, _chunk, 0)

        def _hchunk(j, _):
            base = cid * (EPT * NS) + sid * EPT + j * C
            pltpu.sync_copy(histidx_hbm.at[pl.ds(base, C)], idx_d)
            pltpu.sync_copy(ones1, hist_sh.at[idx_d], add=True)
            return 0

        lax.fori_loop(0, hchunks, _hchunk, 0)
        plsc.subcore_barrier()

        for t in range(NZ):
            pltpu.sync_copy(agg_sh.at[pl.ds(r0 + t * C, C)], rows)
            pltpu.sync_copy(rows, agg_out.at[pl.ds(cid * NP + r0 + t * C, C)])
        pltpu.sync_copy(hist_sh.at[pl.ds(r0, RPT)], hbounce)
        pltpu.sync_copy(hbounce, hist_out.at[pl.ds(cid * NP + r0, RPT)])

    return stage1


@functools.lru_cache(maxsize=None)
def _make_stage2(N, D, NP, EPW):
    nchunks = EPW // C
    RPT = NP // NS
    NZ = RPT // C
    mesh = plsc.VectorSubcoreMesh(core_axis_name="c", subcore_axis_name="s")

    @functools.partial(
        pl.kernel,
        out_type=jax.ShapeDtypeStruct((NC * NP, D), jnp.float32),
        mesh=mesh,
        scratch_types=[
            pltpu.VMEM((C,), jnp.int32),
            pltpu.VMEM((C,), jnp.int32),
            pltpu.VMEM((C,), jnp.int32),
            pltpu.VMEM((C, D), jnp.float32),
            pltpu.VMEM((C, D), jnp.float32),
            pltpu.VMEM_SHARED((NP, D), jnp.float32),
            pltpu.SemaphoreType.DMA,
            pltpu.SemaphoreType.DMA,
        ],
    )
    def stage2(h_hbm, srcg_hbm, dstg_hbm, srcs_hbm,
               s_out,
               idx_a, idx_b, idx_s, rows_a, rows_b,
               s_sh, sem_a, sem_b):
        cid = lax.axis_index("c")
        sid = lax.axis_index("s")
        wid = sid * NC + cid
        zero16 = jnp.zeros((16,), jnp.float32)

        def _zrow(r, _):
            for k in range(D // 16):
                rows_a[r, pl.ds(k * 16, 16)] = zero16
            return 0

        lax.fori_loop(0, C, _zrow, 0)
        r0 = sid * RPT
        for t in range(NZ):
            pltpu.sync_copy(rows_a, s_sh.at[pl.ds(r0 + t * C, C)])
        plsc.subcore_barrier()

        def _chunk(j, _):
            base = wid * EPW + j * C
            c1 = pltpu.async_copy(srcg_hbm.at[pl.ds(base, C)], idx_a, sem_a)
            c2 = pltpu.async_copy(dstg_hbm.at[pl.ds(base, C)], idx_b, sem_b)
            c3 = pltpu.async_copy(srcs_hbm.at[pl.ds(base, C)], idx_s, sem_a)
            c1.wait()
            c2.wait()
            c3.wait()
            ca = pltpu.async_copy(h_hbm.at[idx_a], rows_a, sem_a)
            cb = pltpu.async_copy(h_hbm.at[idx_b], rows_b, sem_b)
            ca.wait()
            cb.wait()

            def _erow(r, _):
                for k in range(D // 16):
                    a = rows_a[r, pl.ds(k * 16, 16)]
                    bb = rows_b[r, pl.ds(k * 16, 16)]
                    d = a - bb
                    rows_a[r, pl.ds(k * 16, 16)] = d * d
                return 0

            lax.fori_loop(0, C, _erow, 0)
            pltpu.sync_copy(rows_a, s_sh.at[idx_s], add=True)
            return 0

        lax.fori_loop(0, nchunks, _chunk, 0)
        plsc.subcore_barrier()
        for t in range(NZ):
            pltpu.sync_copy(s_sh.at[pl.ds(r0 + t * C, C)], rows_a)
            pltpu.sync_copy(rows_a, s_out.at[pl.ds(cid * NP + r0 + t * C, C)])

    return stage2


@functools.lru_cache(maxsize=None)
def _make_conv(N, D, BN):
    def body(x_ref, agg_ref, deg_ref, ws_ref, wn_ref, b_ref, h_ref):
        deg = jnp.maximum(deg_ref[...], 1.0)
        mean = (agg_ref[0] + agg_ref[1]) / deg
        h = jnp.dot(x_ref[...], ws_ref[...], preferred_element_type=jnp.float32)
        h = h + jnp.dot(mean, wn_ref[...], preferred_element_type=jnp.float32)
        h = h + b_ref[...]
        h_ref[...] = jnp.maximum(h, 0.0)

    return pl.pallas_call(
        body,
        grid=(N // BN,),
        in_specs=[
            pl.BlockSpec((BN, D), lambda i: (i, 0)),
            pl.BlockSpec((NC, BN, D), lambda i: (0, i, 0)),
            pl.BlockSpec((BN, 1), lambda i: (i, 0)),
            pl.BlockSpec((D, D), lambda i: (0, 0)),
            pl.BlockSpec((D, D), lambda i: (0, 0)),
            pl.BlockSpec((1, D), lambda i: (0, 0)),
        ],
        out_specs=pl.BlockSpec((BN, D), lambda i: (i, 0)),
        out_shape=jax.ShapeDtypeStruct((N, D), jnp.float32),
    )


@functools.lru_cache(maxsize=None)
def _make_final(N, D, BN):
    def body(s_ref, cnt_ref, gg_ref):
        cnt = jnp.maximum(cnt_ref[...], 1.0)
        gg_ref[...] = jnp.tanh((s_ref[0] + s_ref[1]) / cnt)

    return pl.pallas_call(
        body,
        grid=(N // BN,),
        in_specs=[
            pl.BlockSpec((NC, BN, D), lambda i: (0, i, 0)),
            pl.BlockSpec((BN, 1), lambda i: (i, 0)),
        ],
        out_specs=pl.BlockSpec((BN, D), lambda i: (i, 0)),
        out_shape=jax.ShapeDtypeStruct((N, D), jnp.float32),
    )


def kernel(X, edge_index, W_self, W_neigh, b):
    N, D = X.shape
    E = edge_index.shape[1]
    NP = _ceil_to(N + 1, NS * C)
    E_pad = _ceil_to(E, NW * C)
    EPW = E_pad // NW
    EPT = E_pad // NS

    src = edge_index[0]
    dst = edge_index[1]
    pad = E_pad - E
    zpad = jnp.zeros((pad,), jnp.int32)
    gpad = jnp.full((pad,), N, jnp.int32)
    src_g = jnp.concatenate([src, zpad])
    dst_g = jnp.concatenate([dst, zpad])
    src_s = jnp.concatenate([src, gpad])
    dst_s = jnp.concatenate([dst, gpad])
    hist_idx = jnp.concatenate([dst_s, src_s])

    agg2, hist2 = _make_stage1(N, D, NP, EPW, EPT)(X, src_g, dst_s, hist_idx)
    agg2 = agg2.reshape(NC, NP, D)
    hist2 = hist2.reshape(NC, NP)
    deg_col = hist2[0, :N, None]
    cnt_col = hist2[1, :N, None]
    h = _make_conv(N, D, 400)(X, agg2, deg_col, W_self, W_neigh, b.reshape(1, D))
    s2 = _make_stage2(N, D, NP, EPW)(h, src_g, dst_g, src_s)
    s2 = s2.reshape(NC, NP, D)
    gg = _make_final(N, D, 400)(s2, cnt_col)
    return gg
